# baseline (speedup 1.0000x reference)
import functools

import jax
import jax.numpy as jnp
from jax import lax
from jax.experimental import pallas as pl
from jax.experimental.pallas import tpu as pltpu
from jax.experimental.pallas import tpu_sc as plsc

_B = 4
_N = 4096 * 4096
_RANK = 16693329
_NW = 32
_WPB = _NW // _B
_RPW = 4096 // _WPB
_H1 = 1 << 14
_H2 = 1 << 14
_CROWS = 4
_ROWS = 128
_ABS_MASK = 0x7FFFFFFF


def _zero_vmem(ref, n):
    def body(i, _):
        ref[pl.ds(i * 16, 16)] = jnp.zeros((16,), jnp.int32)
        return 0
    lax.fori_loop(0, n // 16, body, 0)


@functools.cache
def _build_sc_kernels():
    mesh = plsc.VectorSubcoreMesh(core_axis_name="c", subcore_axis_name="s")

    @functools.partial(
        pl.kernel,
        mesh=mesh,
        compiler_params=pltpu.CompilerParams(needs_layout_passes=False),
        out_type=jax.ShapeDtypeStruct((_NW, _H1), jnp.int32),
        scratch_types=[
            pltpu.VMEM((_CROWS, 4096), jnp.float32),
            pltpu.VMEM((_CROWS, 4096), jnp.float32),
            pltpu.VMEM((_H1,), jnp.int32),
            pltpu.VMEM((_H1,), jnp.int32),
            pltpu.VMEM((_H1,), jnp.int32),
            pltpu.VMEM((_H1,), jnp.int32),
            pltpu.SemaphoreType.DMA,
            pltpu.SemaphoreType.DMA,
        ],
    )
    def _sc_hist1(x_hbm, out_hbm, buf0, buf1, hist_a, hist_b, hist_c, hist_d,
                  sem0, sem1):
        wid = lax.axis_index("s") * 2 + lax.axis_index("c")
        bi = wid // _WPB
        row0 = (wid % _WPB) * _RPW
        nch = _RPW // _CROWS
        sems = (sem0, sem1)
        bufs = (buf0, buf1)
        hists = (hist_a, hist_b, hist_c, hist_d)
        for hh in hists:
            _zero_vmem(hh, _H1)
        ones = jnp.ones((16,), jnp.int32)

        def _copy(c, b):
            return pltpu.make_async_copy(
                x_hbm.at[bi, pl.ds(row0 + c * _CROWS, _CROWS), :],
                bufs[b], sems[b])

        for b in range(2):
            _copy(b, b).start()

        def pair_body(g, _):
            for b in range(2):
                c = g * 2 + b
                _copy(c, b).wait()
                bref = bufs[b]
                for r in range(_CROWS):

                    def slice_body(i, _, r=r):
                        for k in range(4):
                            v = bref[r, pl.ds(i * 64 + k * 16, 16)]
                            u = lax.bitcast_convert_type(v, jnp.int32) & jnp.int32(_ABS_MASK)
                            plsc.addupdate_scatter(hists[k], [u >> 17], ones)
                        return 0

                    lax.fori_loop(0, 4096 // 64, slice_body, 0, unroll=8)

                @pl.when(c + 2 < nch)
                def _():
                    _copy(c + 2, b).start()
            return 0

        lax.fori_loop(0, nch // 2, pair_body, 0)

        def merge_body(i, _):
            sl = pl.ds(i * 16, 16)
            hist_a[sl] = ((hist_a[sl] + hist_b[sl]) + (hist_c[sl] + hist_d[sl]))
            return 0

        lax.fori_loop(0, _H1 // 16, merge_body, 0, unroll=8)
        pltpu.sync_copy(hist_a, out_hbm.at[wid])

    @functools.partial(
        pl.kernel,
        mesh=mesh,
        compiler_params=pltpu.CompilerParams(needs_layout_passes=False),
        out_type=jax.ShapeDtypeStruct((_NW, _H2), jnp.int32),
        scratch_types=[
            pltpu.VMEM((_CROWS, 4096), jnp.float32),
            pltpu.VMEM((_CROWS, 4096), jnp.float32),
            pltpu.VMEM((_H2,), jnp.int32),
            pltpu.VMEM((_H2,), jnp.int32),
            pltpu.VMEM((_H2,), jnp.int32),
            pltpu.VMEM((_H2,), jnp.int32),
            pltpu.VMEM((16,), jnp.int32),
            pltpu.SemaphoreType.DMA,
            pltpu.SemaphoreType.DMA,
        ],
    )
    def _sc_hist2(x_hbm, tsel_hbm, out_hbm, buf0, buf1, hist_a, hist_b,
                  hist_c, hist_d, tbuf, sem0, sem1):
        wid = lax.axis_index("s") * 2 + lax.axis_index("c")
        bi = wid // _WPB
        row0 = (wid % _WPB) * _RPW
        nch = _RPW // _CROWS
        sems = (sem0, sem1)
        bufs = (buf0, buf1)
        pltpu.sync_copy(tsel_hbm, tbuf)
        tvec = plsc.load_gather(tbuf, [jnp.zeros((16,), jnp.int32) + bi])
        hists = (hist_a, hist_b, hist_c, hist_d)
        for hh in hists:
            _zero_vmem(hh, _H2)
        ones = jnp.ones((16,), jnp.int32)

        def _copy(c, b):
            return pltpu.make_async_copy(
                x_hbm.at[bi, pl.ds(row0 + c * _CROWS, _CROWS), :],
                bufs[b], sems[b])

        for b in range(2):
            _copy(b, b).start()

        def pair_body(g, _):
            for b in range(2):
                c = g * 2 + b
                _copy(c, b).wait()
                bref = bufs[b]
                for r in range(_CROWS):

                    def slice_body(i, _, r=r):
                        for k in range(4):
                            v = bref[r, pl.ds(i * 64 + k * 16, 16)]
                            u = lax.bitcast_convert_type(v, jnp.int32) & jnp.int32(_ABS_MASK)
                            mk = (u >> 17) == tvec
                            plsc.addupdate_scatter(
                                hists[k], [(u & jnp.int32(0x1FFFF)) >> 3], ones,
                                mask=mk)
                        return 0

                    lax.fori_loop(0, 4096 // 64, slice_body, 0, unroll=8)

                @pl.when(c + 2 < nch)
                def _():
                    _copy(c + 2, b).start()
            return 0

        lax.fori_loop(0, nch // 2, pair_body, 0)

        def merge_body(i, _):
            sl = pl.ds(i * 16, 16)
            hist_a[sl] = ((hist_a[sl] + hist_b[sl]) + (hist_c[sl] + hist_d[sl]))
            return 0

        lax.fori_loop(0, _H2 // 16, merge_body, 0, unroll=8)
        pltpu.sync_copy(hist_a, out_hbm.at[wid])

    return _sc_hist1, _sc_hist2


def _cumsum_last(a):
    n = a.shape[-1]
    s = 1
    while s < n:
        a = a + jnp.concatenate(
            [jnp.zeros(a.shape[:-1] + (s,), a.dtype), a[..., :-s]], axis=-1)
        s *= 2
    return a


def _cumsum_rows(a):
    n = a.shape[0]
    s = 1
    while s < n:
        a = a + jnp.concatenate(
            [jnp.zeros((s,) + a.shape[1:], a.dtype), a[:-s]], axis=0)
        s *= 2
    return a


def _rank_select(h_rows, rank):
    r_dim = h_rows.shape[0]
    cw = _cumsum_last(h_rows)
    cw_excl = cw - h_rows
    rt = jnp.broadcast_to(cw[:, -1:], (r_dim, 128))
    pref = _cumsum_rows(rt) - rt
    cb = pref + cw_excl
    m = (cb <= rank).astype(jnp.int32)
    t = jnp.sum(m) - 1
    cb_t = jnp.max(m * cb)
    return t, cb_t


def _sel1_body(h_ref, o_ref):
    h = jnp.sum(h_ref[...], axis=0)
    t, cb_t = _rank_select(h.reshape(_H1 // 128, 128), jnp.int32(_RANK))
    lane = lax.broadcasted_iota(jnp.int32, (1, 1, 128), 2)
    r_i = jnp.int32(_RANK) - cb_t
    o_ref[...] = jnp.where(lane == 0, t, jnp.where(lane == 1, r_i, 0))


def _sel2_body(h_ref, s1_ref, o_ref):
    h = jnp.sum(h_ref[...], axis=0)
    t1 = s1_ref[0, 0, 0]
    rank = s1_ref[0, 0, 1]
    t2, _ = _rank_select(h.reshape(_H2 // 128, 128), rank)
    qbits = (t1 << 17) | (t2 << 3)
    qv = lax.bitcast_convert_type(qbits, jnp.float32)
    scale = jnp.float32(3.0) / jnp.maximum(qv, jnp.float32(3.0))
    o_ref[...] = jnp.full((1, 1, 128), scale, jnp.float32)


def _scale_body(scale_ref, x_ref, o_ref):
    b = pl.program_id(0)
    o_ref[...] = x_ref[...] * scale_ref[b]


def kernel(x):
    sc_hist1, sc_hist2 = _build_sc_kernels()

    h1 = sc_hist1(x)

    sel1 = pl.pallas_call(
        _sel1_body,
        grid=(_B,),
        in_specs=[pl.BlockSpec((_WPB, _H1), lambda b: (b, 0))],
        out_specs=pl.BlockSpec((1, 1, 128), lambda b: (b, 0, 0)),
        out_shape=jax.ShapeDtypeStruct((_B, 1, 128), jnp.int32),
    )(h1)

    tsel = jnp.zeros((16,), jnp.int32).at[:_B].set(sel1[:, 0, 0])
    h2 = sc_hist2(x, tsel)

    sel2 = pl.pallas_call(
        _sel2_body,
        grid=(_B,),
        in_specs=[
            pl.BlockSpec((_WPB, _H2), lambda b: (b, 0)),
            pl.BlockSpec((1, 1, 128), lambda b: (b, 0, 0)),
        ],
        out_specs=pl.BlockSpec((1, 1, 128), lambda b: (b, 0, 0)),
        out_shape=jax.ShapeDtypeStruct((_B, 1, 128), jnp.float32),
    )(h2, sel1)

    scale = sel2[:, 0, 0]

    out = pl.pallas_call(
        _scale_body,
        grid=(_B, x.shape[1] // _ROWS),
        in_specs=[
            pl.BlockSpec(memory_space=pltpu.SMEM),
            pl.BlockSpec((1, _ROWS, 4096), lambda b, c: (b, c, 0)),
        ],
        out_specs=pl.BlockSpec((1, _ROWS, 4096), lambda b, c: (b, c, 0)),
        out_shape=jax.ShapeDtypeStruct(x.shape, x.dtype),
    )(scale, x)
    return out


# speedup vs baseline: 1.6720x; 1.6720x over previous
import functools

import jax
import jax.numpy as jnp
from jax import lax
from jax.experimental import pallas as pl
from jax.experimental.pallas import tpu as pltpu
from jax.experimental.pallas import tpu_sc as plsc

_B = 4
_N = 4096 * 4096
_RANK = 16693329
_NW = 32
_WPB = _NW // _B
_RPW = 4096 // _WPB
_H1 = 1 << 15
_H2 = 1 << 15
_CROWS = 4
_ROWS = 128
_ABS_MASK = 0x7FFFFFFF


def _zero_vmem(ref, n):
    def body(i, _):
        ref[pl.ds(i * 16, 16)] = jnp.zeros((16,), jnp.int32)
        return 0
    lax.fori_loop(0, n // 16, body, 0)


@functools.cache
def _build_sc_kernels():
    mesh = plsc.VectorSubcoreMesh(core_axis_name="c", subcore_axis_name="s")

    @functools.partial(
        pl.kernel,
        mesh=mesh,
        compiler_params=pltpu.CompilerParams(needs_layout_passes=False),
        out_type=jax.ShapeDtypeStruct((_NW, _H1), jnp.int32),
        scratch_types=[
            pltpu.VMEM((_CROWS, 4096), jnp.float32),
            pltpu.VMEM((_CROWS, 4096), jnp.float32),
            pltpu.VMEM((_H1,), jnp.int32),
            pltpu.VMEM((_H1,), jnp.int32),
            pltpu.SemaphoreType.DMA,
            pltpu.SemaphoreType.DMA,
        ],
    )
    def _sc_hist1(x_hbm, out_hbm, buf0, buf1, hist_a, hist_b, sem0, sem1):
        wid = lax.axis_index("s") * 2 + lax.axis_index("c")
        bi = wid // _WPB
        row0 = (wid % _WPB) * _RPW
        nch = _RPW // _CROWS
        sems = (sem0, sem1)
        bufs = (buf0, buf1)
        _zero_vmem(hist_a, _H1)
        _zero_vmem(hist_b, _H1)
        ones = jnp.ones((16,), jnp.int32)

        def _copy(c, b):
            return pltpu.make_async_copy(
                x_hbm.at[bi, pl.ds(row0 + c * _CROWS, _CROWS), :],
                bufs[b], sems[b])

        for b in range(2):
            _copy(b, b).start()

        def pair_body(g, _):
            for b in range(2):
                c = g * 2 + b
                _copy(c, b).wait()
                bref = bufs[b]
                for r in range(_CROWS):

                    def slice_body(i, _, r=r):
                        v0 = bref[r, pl.ds(i * 32, 16)]
                        v1 = bref[r, pl.ds(i * 32 + 16, 16)]
                        u0 = lax.bitcast_convert_type(v0, jnp.int32) & jnp.int32(_ABS_MASK)
                        u1 = lax.bitcast_convert_type(v1, jnp.int32) & jnp.int32(_ABS_MASK)
                        plsc.addupdate_scatter(hist_a, [u0 >> 16], ones)
                        plsc.addupdate_scatter(hist_b, [u1 >> 16], ones)
                        return 0

                    lax.fori_loop(0, 4096 // 32, slice_body, 0, unroll=8)

                @pl.when(c + 2 < nch)
                def _():
                    _copy(c + 2, b).start()
            return 0

        lax.fori_loop(0, nch // 2, pair_body, 0)

        def merge_body(i, _):
            sl = pl.ds(i * 16, 16)
            hist_a[sl] = hist_a[sl] + hist_b[sl]
            return 0

        lax.fori_loop(0, _H1 // 16, merge_body, 0, unroll=8)
        pltpu.sync_copy(hist_a, out_hbm.at[wid])

    @functools.partial(
        pl.kernel,
        mesh=mesh,
        compiler_params=pltpu.CompilerParams(needs_layout_passes=False),
        out_type=jax.ShapeDtypeStruct((_NW, _H2), jnp.int32),
        scratch_types=[
            pltpu.VMEM((_CROWS, 4096), jnp.float32),
            pltpu.VMEM((_CROWS, 4096), jnp.float32),
            pltpu.VMEM((_H2,), jnp.int32),
            pltpu.VMEM((_H2,), jnp.int32),
            pltpu.VMEM((16,), jnp.int32),
            pltpu.SemaphoreType.DMA,
            pltpu.SemaphoreType.DMA,
        ],
    )
    def _sc_hist2(x_hbm, tsel_hbm, out_hbm, buf0, buf1, hist_a, hist_b, tbuf,
                  sem0, sem1):
        wid = lax.axis_index("s") * 2 + lax.axis_index("c")
        bi = wid // _WPB
        row0 = (wid % _WPB) * _RPW
        nch = _RPW // _CROWS
        sems = (sem0, sem1)
        bufs = (buf0, buf1)
        pltpu.sync_copy(tsel_hbm, tbuf)
        tvec = plsc.load_gather(tbuf, [jnp.zeros((16,), jnp.int32) + bi])
        _zero_vmem(hist_a, _H2)
        _zero_vmem(hist_b, _H2)
        ones = jnp.ones((16,), jnp.int32)

        def _copy(c, b):
            return pltpu.make_async_copy(
                x_hbm.at[bi, pl.ds(row0 + c * _CROWS, _CROWS), :],
                bufs[b], sems[b])

        for b in range(2):
            _copy(b, b).start()

        def pair_body(g, _):
            for b in range(2):
                c = g * 2 + b
                _copy(c, b).wait()
                bref = bufs[b]
                for r in range(_CROWS):

                    def slice_body(i, _, r=r):
                        v0 = bref[r, pl.ds(i * 32, 16)]
                        v1 = bref[r, pl.ds(i * 32 + 16, 16)]
                        u0 = lax.bitcast_convert_type(v0, jnp.int32) & jnp.int32(_ABS_MASK)
                        u1 = lax.bitcast_convert_type(v1, jnp.int32) & jnp.int32(_ABS_MASK)
                        m0 = (u0 >> 16) == tvec
                        m1 = (u1 >> 16) == tvec
                        plsc.addupdate_scatter(
                            hist_a, [(u0 & jnp.int32(0xFFFF)) >> 1], ones, mask=m0)
                        plsc.addupdate_scatter(
                            hist_b, [(u1 & jnp.int32(0xFFFF)) >> 1], ones, mask=m1)
                        return 0

                    lax.fori_loop(0, 4096 // 32, slice_body, 0, unroll=8)

                @pl.when(c + 2 < nch)
                def _():
                    _copy(c + 2, b).start()
            return 0

        lax.fori_loop(0, nch // 2, pair_body, 0)

        def merge_body(i, _):
            sl = pl.ds(i * 16, 16)
            hist_a[sl] = hist_a[sl] + hist_b[sl]
            return 0

        lax.fori_loop(0, _H2 // 16, merge_body, 0, unroll=8)
        pltpu.sync_copy(hist_a, out_hbm.at[wid])

    return _sc_hist1, _sc_hist2


def _cumsum_last(a):
    n = a.shape[-1]
    s = 1
    while s < n:
        a = a + jnp.concatenate(
            [jnp.zeros(a.shape[:-1] + (s,), a.dtype), a[..., :-s]], axis=-1)
        s *= 2
    return a


def _cumsum_rows(a):
    n = a.shape[0]
    s = 1
    while s < n:
        a = a + jnp.concatenate(
            [jnp.zeros((s,) + a.shape[1:], a.dtype), a[:-s]], axis=0)
        s *= 2
    return a


def _rank_select(h_rows, rank):
    r_dim = h_rows.shape[0]
    cw = _cumsum_last(h_rows)
    cw_excl = cw - h_rows
    rt = jnp.broadcast_to(cw[:, -1:], (r_dim, 128))
    pref = _cumsum_rows(rt) - rt
    cb = pref + cw_excl
    m = (cb <= rank).astype(jnp.int32)
    t = jnp.sum(m) - 1
    cb_t = jnp.max(m * cb)
    return t, cb_t


def _sel1_body(h_ref, o_ref):
    h = jnp.sum(h_ref[...], axis=0)
    t, cb_t = _rank_select(h.reshape(_H1 // 128, 128), jnp.int32(_RANK))
    lane = lax.broadcasted_iota(jnp.int32, (1, 1, 128), 2)
    r_i = jnp.int32(_RANK) - cb_t
    o_ref[...] = jnp.where(lane == 0, t, jnp.where(lane == 1, r_i, 0))


def _sel2_body(h_ref, s1_ref, o_ref):
    h = jnp.sum(h_ref[...], axis=0)
    t1 = s1_ref[0, 0, 0]
    rank = s1_ref[0, 0, 1]
    t2, _ = _rank_select(h.reshape(_H2 // 128, 128), rank)
    qbits = (t1 << 16) | (t2 << 1)
    qv = lax.bitcast_convert_type(qbits, jnp.float32)
    scale = jnp.float32(3.0) / jnp.maximum(qv, jnp.float32(3.0))
    o_ref[...] = jnp.full((1, 1, 128), scale, jnp.float32)


def _scale_body(scale_ref, x_ref, o_ref):
    b = pl.program_id(0)
    o_ref[...] = x_ref[...] * scale_ref[b]


def kernel(x):
    sc_hist1, sc_hist2 = _build_sc_kernels()

    h1 = sc_hist1(x)

    sel1 = pl.pallas_call(
        _sel1_body,
        grid=(_B,),
        in_specs=[pl.BlockSpec((_WPB, _H1), lambda b: (b, 0))],
        out_specs=pl.BlockSpec((1, 1, 128), lambda b: (b, 0, 0)),
        out_shape=jax.ShapeDtypeStruct((_B, 1, 128), jnp.int32),
    )(h1)

    tsel = jnp.zeros((16,), jnp.int32).at[:_B].set(sel1[:, 0, 0])
    h2 = sc_hist2(x, tsel)

    sel2 = pl.pallas_call(
        _sel2_body,
        grid=(_B,),
        in_specs=[
            pl.BlockSpec((_WPB, _H2), lambda b: (b, 0)),
            pl.BlockSpec((1, 1, 128), lambda b: (b, 0, 0)),
        ],
        out_specs=pl.BlockSpec((1, 1, 128), lambda b: (b, 0, 0)),
        out_shape=jax.ShapeDtypeStruct((_B, 1, 128), jnp.float32),
    )(h2, sel1)

    scale = sel2[:, 0, 0]

    out = pl.pallas_call(
        _scale_body,
        grid=(_B, x.shape[1] // _ROWS),
        in_specs=[
            pl.BlockSpec(memory_space=pltpu.SMEM),
            pl.BlockSpec((1, _ROWS, 4096), lambda b, c: (b, c, 0)),
        ],
        out_specs=pl.BlockSpec((1, _ROWS, 4096), lambda b, c: (b, c, 0)),
        out_shape=jax.ShapeDtypeStruct(x.shape, x.dtype),
    )(scale, x)
    return out


# speedup vs baseline: 1.6725x; 1.0003x over previous
import functools

import jax
import jax.numpy as jnp
from jax import lax
from jax.experimental import pallas as pl
from jax.experimental.pallas import tpu as pltpu
from jax.experimental.pallas import tpu_sc as plsc

_B = 4
_N = 4096 * 4096
_RANK = 16693329
_NW = 32
_WPB = _NW // _B
_RPW = 4096 // _WPB
_H1 = 1 << 15
_H2 = 1 << 15
_CROWS = 4
_ROWS = 128
_ABS_MASK = 0x7FFFFFFF


def _zero_vmem(ref, n):
    def body(i, _):
        ref[pl.ds(i * 16, 16)] = jnp.zeros((16,), jnp.int32)
        return 0
    lax.fori_loop(0, n // 16, body, 0)


@functools.cache
def _build_sc_kernels():
    mesh = plsc.VectorSubcoreMesh(core_axis_name="c", subcore_axis_name="s")

    @functools.partial(
        pl.kernel,
        mesh=mesh,
        compiler_params=pltpu.CompilerParams(needs_layout_passes=False),
        out_type=jax.ShapeDtypeStruct((_NW, _H1), jnp.int32),
        scratch_types=[
            pltpu.VMEM((_CROWS, 4096), jnp.float32),
            pltpu.VMEM((_CROWS, 4096), jnp.float32),
            pltpu.VMEM((_H1,), jnp.int32),
            pltpu.VMEM((_H1,), jnp.int32),
            pltpu.SemaphoreType.DMA,
            pltpu.SemaphoreType.DMA,
        ],
    )
    def _sc_hist1(x_hbm, out_hbm, buf0, buf1, hist_a, hist_b, sem0, sem1):
        wid = lax.axis_index("s") * 2 + lax.axis_index("c")
        bi = wid // _WPB
        row0 = (wid % _WPB) * _RPW
        nch = _RPW // _CROWS
        sems = (sem0, sem1)
        bufs = (buf0, buf1)
        _zero_vmem(hist_a, _H1)
        _zero_vmem(hist_b, _H1)
        ones = jnp.ones((16,), jnp.int32)

        def _copy(c, b):
            return pltpu.make_async_copy(
                x_hbm.at[bi, pl.ds(row0 + c * _CROWS, _CROWS), :],
                bufs[b], sems[b])

        for b in range(2):
            _copy(b, b).start()

        def pair_body(g, _):
            for b in range(2):
                c = g * 2 + b
                _copy(c, b).wait()
                bref = bufs[b]
                for r in range(_CROWS):

                    def slice_body(i, _, r=r):
                        v0 = bref[r, pl.ds(i * 32, 16)]
                        v1 = bref[r, pl.ds(i * 32 + 16, 16)]
                        u0 = lax.bitcast_convert_type(v0, jnp.int32) & jnp.int32(_ABS_MASK)
                        u1 = lax.bitcast_convert_type(v1, jnp.int32) & jnp.int32(_ABS_MASK)
                        plsc.addupdate_scatter(hist_a, [u0 >> 16], ones)
                        plsc.addupdate_scatter(hist_b, [u1 >> 16], ones)
                        return 0

                    lax.fori_loop(0, 4096 // 32, slice_body, 0, unroll=16)

                @pl.when(c + 2 < nch)
                def _():
                    _copy(c + 2, b).start()
            return 0

        lax.fori_loop(0, nch // 2, pair_body, 0)

        def merge_body(i, _):
            sl = pl.ds(i * 16, 16)
            hist_a[sl] = hist_a[sl] + hist_b[sl]
            return 0

        lax.fori_loop(0, _H1 // 16, merge_body, 0, unroll=8)
        pltpu.sync_copy(hist_a, out_hbm.at[wid])

    @functools.partial(
        pl.kernel,
        mesh=mesh,
        compiler_params=pltpu.CompilerParams(needs_layout_passes=False),
        out_type=jax.ShapeDtypeStruct((_NW, _H2), jnp.int32),
        scratch_types=[
            pltpu.VMEM((_CROWS, 4096), jnp.float32),
            pltpu.VMEM((_CROWS, 4096), jnp.float32),
            pltpu.VMEM((_H2,), jnp.int32),
            pltpu.VMEM((_H2,), jnp.int32),
            pltpu.VMEM((16,), jnp.int32),
            pltpu.SemaphoreType.DMA,
            pltpu.SemaphoreType.DMA,
        ],
    )
    def _sc_hist2(x_hbm, tsel_hbm, out_hbm, buf0, buf1, hist_a, hist_b, tbuf,
                  sem0, sem1):
        wid = lax.axis_index("s") * 2 + lax.axis_index("c")
        bi = wid // _WPB
        row0 = (wid % _WPB) * _RPW
        nch = _RPW // _CROWS
        sems = (sem0, sem1)
        bufs = (buf0, buf1)
        pltpu.sync_copy(tsel_hbm, tbuf)
        tvec = plsc.load_gather(tbuf, [jnp.zeros((16,), jnp.int32) + bi])
        _zero_vmem(hist_a, _H2)
        _zero_vmem(hist_b, _H2)
        ones = jnp.ones((16,), jnp.int32)

        def _copy(c, b):
            return pltpu.make_async_copy(
                x_hbm.at[bi, pl.ds(row0 + c * _CROWS, _CROWS), :],
                bufs[b], sems[b])

        for b in range(2):
            _copy(b, b).start()

        def pair_body(g, _):
            for b in range(2):
                c = g * 2 + b
                _copy(c, b).wait()
                bref = bufs[b]
                for r in range(_CROWS):

                    def slice_body(i, _, r=r):
                        v0 = bref[r, pl.ds(i * 32, 16)]
                        v1 = bref[r, pl.ds(i * 32 + 16, 16)]
                        u0 = lax.bitcast_convert_type(v0, jnp.int32) & jnp.int32(_ABS_MASK)
                        u1 = lax.bitcast_convert_type(v1, jnp.int32) & jnp.int32(_ABS_MASK)
                        m0 = (u0 >> 16) == tvec
                        m1 = (u1 >> 16) == tvec
                        plsc.addupdate_scatter(
                            hist_a, [(u0 & jnp.int32(0xFFFF)) >> 1], ones, mask=m0)
                        plsc.addupdate_scatter(
                            hist_b, [(u1 & jnp.int32(0xFFFF)) >> 1], ones, mask=m1)
                        return 0

                    lax.fori_loop(0, 4096 // 32, slice_body, 0, unroll=16)

                @pl.when(c + 2 < nch)
                def _():
                    _copy(c + 2, b).start()
            return 0

        lax.fori_loop(0, nch // 2, pair_body, 0)

        def merge_body(i, _):
            sl = pl.ds(i * 16, 16)
            hist_a[sl] = hist_a[sl] + hist_b[sl]
            return 0

        lax.fori_loop(0, _H2 // 16, merge_body, 0, unroll=8)
        pltpu.sync_copy(hist_a, out_hbm.at[wid])

    return _sc_hist1, _sc_hist2


def _cumsum_last(a):
    n = a.shape[-1]
    s = 1
    while s < n:
        a = a + jnp.concatenate(
            [jnp.zeros(a.shape[:-1] + (s,), a.dtype), a[..., :-s]], axis=-1)
        s *= 2
    return a


def _cumsum_rows(a):
    n = a.shape[0]
    s = 1
    while s < n:
        a = a + jnp.concatenate(
            [jnp.zeros((s,) + a.shape[1:], a.dtype), a[:-s]], axis=0)
        s *= 2
    return a


def _rank_select(h_rows, rank):
    r_dim = h_rows.shape[0]
    cw = _cumsum_last(h_rows)
    cw_excl = cw - h_rows
    rt = jnp.broadcast_to(cw[:, -1:], (r_dim, 128))
    pref = _cumsum_rows(rt) - rt
    cb = pref + cw_excl
    m = (cb <= rank).astype(jnp.int32)
    t = jnp.sum(m) - 1
    cb_t = jnp.max(m * cb)
    return t, cb_t


def _sel1_body(h_ref, o_ref):
    h = jnp.sum(h_ref[...], axis=0)
    t, cb_t = _rank_select(h.reshape(_H1 // 128, 128), jnp.int32(_RANK))
    lane = lax.broadcasted_iota(jnp.int32, (1, 1, 128), 2)
    r_i = jnp.int32(_RANK) - cb_t
    o_ref[...] = jnp.where(lane == 0, t, jnp.where(lane == 1, r_i, 0))


def _sel2_body(h_ref, s1_ref, o_ref):
    h = jnp.sum(h_ref[...], axis=0)
    t1 = s1_ref[0, 0, 0]
    rank = s1_ref[0, 0, 1]
    t2, _ = _rank_select(h.reshape(_H2 // 128, 128), rank)
    qbits = (t1 << 16) | (t2 << 1)
    qv = lax.bitcast_convert_type(qbits, jnp.float32)
    scale = jnp.float32(3.0) / jnp.maximum(qv, jnp.float32(3.0))
    o_ref[...] = jnp.full((1, 1, 128), scale, jnp.float32)


def _scale_body(scale_ref, x_ref, o_ref):
    b = pl.program_id(0)
    o_ref[...] = x_ref[...] * scale_ref[b]


def kernel(x):
    sc_hist1, sc_hist2 = _build_sc_kernels()

    h1 = sc_hist1(x)

    sel1 = pl.pallas_call(
        _sel1_body,
        grid=(_B,),
        in_specs=[pl.BlockSpec((_WPB, _H1), lambda b: (b, 0))],
        out_specs=pl.BlockSpec((1, 1, 128), lambda b: (b, 0, 0)),
        out_shape=jax.ShapeDtypeStruct((_B, 1, 128), jnp.int32),
    )(h1)

    tsel = jnp.zeros((16,), jnp.int32).at[:_B].set(sel1[:, 0, 0])
    h2 = sc_hist2(x, tsel)

    sel2 = pl.pallas_call(
        _sel2_body,
        grid=(_B,),
        in_specs=[
            pl.BlockSpec((_WPB, _H2), lambda b: (b, 0)),
            pl.BlockSpec((1, 1, 128), lambda b: (b, 0, 0)),
        ],
        out_specs=pl.BlockSpec((1, 1, 128), lambda b: (b, 0, 0)),
        out_shape=jax.ShapeDtypeStruct((_B, 1, 128), jnp.float32),
    )(h2, sel1)

    scale = sel2[:, 0, 0]

    out = pl.pallas_call(
        _scale_body,
        grid=(_B, x.shape[1] // _ROWS),
        in_specs=[
            pl.BlockSpec(memory_space=pltpu.SMEM),
            pl.BlockSpec((1, _ROWS, 4096), lambda b, c: (b, c, 0)),
        ],
        out_specs=pl.BlockSpec((1, _ROWS, 4096), lambda b, c: (b, c, 0)),
        out_shape=jax.ShapeDtypeStruct(x.shape, x.dtype),
    )(scale, x)
    return out
